# initial kernel scaffold (unmeasured)
import jax
import jax.numpy as jnp
from jax import lax
from jax.experimental import pallas as pl
from jax.experimental.pallas import tpu as pltpu

N_DEV = 8


def kernel(x, Win0, Wout0, Win1, Wout1, Win2, Wout2):
    B, D = x.shape
    H = Win0.shape[1]
    C = H // N_DEV
    f32 = jnp.float32
    bf16 = jnp.bfloat16

    def body(x_ref, win0, wout0, win1, wout1, win2, wout2, out_ref,
             xbuf, win_tile, wout_tile, acc, acc2, hmine,
             rsbuf, agbuf, sendbuf,
             copy_sem, rs_send_sems, rs_recv_sems, ag_send_sems, ag_recv_sems):
        my_i = lax.axis_index("i")

        xbuf[...] = x_ref[...].astype(bf16)

        for l, (win, wout) in enumerate(
            [(win0, wout0), (win1, wout1), (win2, wout2)]
        ):
            for j in range(N_DEV):
                cp = pltpu.make_async_copy(
                    win.at[:, pl.ds(j * C, C)], win_tile, copy_sem
                )
                cp.start()
                cp.wait()
                chunk = jnp.dot(
                    xbuf[...], win_tile[...].astype(bf16),
                    preferred_element_type=f32,
                )

                @pl.when(j == my_i)
                def _():
                    acc[...] = chunk

                @pl.when(j != my_i)
                def _():
                    sendbuf[l, j] = chunk.astype(bf16)
                    rdma = pltpu.make_async_remote_copy(
                        src_ref=sendbuf.at[l, j],
                        dst_ref=rsbuf.at[l, my_i],
                        send_sem=rs_send_sems.at[l, j],
                        recv_sem=rs_recv_sems.at[l, my_i],
                        device_id=j,
                        device_id_type=pl.DeviceIdType.LOGICAL,
                    )
                    rdma.start()

            for j in range(N_DEV):
                @pl.when(j != my_i)
                def _():
                    recv = pltpu.make_async_remote_copy(
                        src_ref=sendbuf.at[l, j],
                        dst_ref=rsbuf.at[l, j],
                        send_sem=rs_send_sems.at[l, j],
                        recv_sem=rs_recv_sems.at[l, j],
                        device_id=j,
                        device_id_type=pl.DeviceIdType.LOGICAL,
                    )
                    recv.wait_recv()
                    acc[...] += rsbuf[l, j].astype(f32)

            hmine[...] = jnp.maximum(acc[...], 0.0).astype(bf16)

            cp = pltpu.make_async_copy(hmine, agbuf.at[l, my_i], copy_sem)
            cp.start()
            cp.wait()
            for j in range(N_DEV):
                @pl.when(j != my_i)
                def _():
                    rdma = pltpu.make_async_remote_copy(
                        src_ref=hmine,
                        dst_ref=agbuf.at[l, my_i],
                        send_sem=ag_send_sems.at[l, j],
                        recv_sem=ag_recv_sems.at[l, my_i],
                        device_id=j,
                        device_id_type=pl.DeviceIdType.LOGICAL,
                    )
                    rdma.start()

            acc2[...] = jnp.zeros((B, D), f32)
            for j in range(N_DEV):
                cp = pltpu.make_async_copy(
                    wout.at[pl.ds(j * C, C), :], wout_tile, copy_sem
                )
                cp.start()
                cp.wait()

                @pl.when(j != my_i)
                def _():
                    recv = pltpu.make_async_remote_copy(
                        src_ref=hmine,
                        dst_ref=agbuf.at[l, j],
                        send_sem=ag_send_sems.at[l, j],
                        recv_sem=ag_recv_sems.at[l, j],
                        device_id=j,
                        device_id_type=pl.DeviceIdType.LOGICAL,
                    )
                    recv.wait_recv()

                acc2[...] += jnp.dot(
                    agbuf[l, j], wout_tile[...].astype(bf16),
                    preferred_element_type=f32,
                )

            if l < 2:
                xbuf[...] = acc2[...].astype(bf16)
            else:
                out_ref[...] = acc2[...]

            for j in range(N_DEV):
                @pl.when(j != my_i)
                def _():
                    s = pltpu.make_async_remote_copy(
                        src_ref=sendbuf.at[l, j],
                        dst_ref=rsbuf.at[l, my_i],
                        send_sem=rs_send_sems.at[l, j],
                        recv_sem=rs_recv_sems.at[l, my_i],
                        device_id=j,
                        device_id_type=pl.DeviceIdType.LOGICAL,
                    )
                    s.wait_send()
                    s2 = pltpu.make_async_remote_copy(
                        src_ref=hmine,
                        dst_ref=agbuf.at[l, my_i],
                        send_sem=ag_send_sems.at[l, j],
                        recv_sem=ag_recv_sems.at[l, my_i],
                        device_id=j,
                        device_id_type=pl.DeviceIdType.LOGICAL,
                    )
                    s2.wait_send()

    return pl.pallas_call(
        body,
        out_shape=jax.ShapeDtypeStruct((B, D), f32),
        in_specs=[
            pl.BlockSpec(memory_space=pltpu.VMEM),
            pl.BlockSpec(memory_space=pltpu.ANY),
            pl.BlockSpec(memory_space=pltpu.ANY),
            pl.BlockSpec(memory_space=pltpu.ANY),
            pl.BlockSpec(memory_space=pltpu.ANY),
            pl.BlockSpec(memory_space=pltpu.ANY),
            pl.BlockSpec(memory_space=pltpu.ANY),
        ],
        out_specs=pl.BlockSpec(memory_space=pltpu.VMEM),
        scratch_shapes=[
            pltpu.VMEM((B, D), bf16),
            pltpu.VMEM((D, C), f32),
            pltpu.VMEM((C, D), f32),
            pltpu.VMEM((B, C), f32),
            pltpu.VMEM((B, D), f32),
            pltpu.VMEM((B, C), bf16),
            pltpu.VMEM((3, N_DEV, B, C), bf16),
            pltpu.VMEM((3, N_DEV, B, C), bf16),
            pltpu.VMEM((3, N_DEV, B, C), bf16),
            pltpu.SemaphoreType.DMA,
            pltpu.SemaphoreType.DMA((3, N_DEV)),
            pltpu.SemaphoreType.DMA((3, N_DEV)),
            pltpu.SemaphoreType.DMA((3, N_DEV)),
            pltpu.SemaphoreType.DMA((3, N_DEV)),
        ],
        compiler_params=pltpu.CompilerParams(collective_id=0),
    )(x, Win0, Wout0, Win1, Wout1, Win2, Wout2)


# baseline (device time: 137960 ns/iter reference)
import jax
import jax.numpy as jnp
from jax import lax
from jax.experimental import pallas as pl
from jax.experimental.pallas import tpu as pltpu

N_DEV = 8


def kernel(x, Win0, Wout0, Win1, Wout1, Win2, Wout2):
    B, D = x.shape
    H = Win0.shape[1]
    C = H // N_DEV
    f32 = jnp.float32
    bf16 = jnp.bfloat16

    def body(x_ref, win0, wout0, win1, wout1, win2, wout2, out_ref,
             xbuf, win_tile, wout_tile, acc, acc2, hmine,
             rsbuf, agbuf, sendbuf,
             copy_sem, rs_send_sems, rs_recv_sems, ag_send_sems, ag_recv_sems):
        my_i = lax.axis_index("i")

        xbuf[...] = x_ref[...].astype(bf16)

        for l, (win, wout) in enumerate(
            [(win0, wout0), (win1, wout1), (win2, wout2)]
        ):
            for j in range(N_DEV):
                cp = pltpu.make_async_copy(
                    win.at[:, pl.ds(j * C, C)], win_tile, copy_sem
                )
                cp.start()
                cp.wait()
                chunk = jnp.dot(
                    xbuf[...], win_tile[...].astype(bf16),
                    preferred_element_type=f32,
                )

                @pl.when(j == my_i)
                def _():
                    acc[...] = chunk

                @pl.when(j != my_i)
                def _():
                    sendbuf[l, j] = chunk.astype(bf16)
                    rdma = pltpu.make_async_remote_copy(
                        src_ref=sendbuf.at[l, j],
                        dst_ref=rsbuf.at[l, my_i],
                        send_sem=rs_send_sems.at[l, j],
                        recv_sem=rs_recv_sems.at[l, my_i],
                        device_id=j,
                        device_id_type=pl.DeviceIdType.LOGICAL,
                    )
                    rdma.start()

            for j in range(N_DEV):
                @pl.when(j != my_i)
                def _():
                    recv = pltpu.make_async_remote_copy(
                        src_ref=sendbuf.at[l, j],
                        dst_ref=rsbuf.at[l, j],
                        send_sem=rs_send_sems.at[l, j],
                        recv_sem=rs_recv_sems.at[l, j],
                        device_id=j,
                        device_id_type=pl.DeviceIdType.LOGICAL,
                    )
                    recv.wait_recv()
                    acc[...] += rsbuf[l, j].astype(f32)

            hmine[...] = jnp.maximum(acc[...], 0.0).astype(bf16)

            cp = pltpu.make_async_copy(hmine, agbuf.at[l, my_i], copy_sem)
            cp.start()
            cp.wait()
            for j in range(N_DEV):
                @pl.when(j != my_i)
                def _():
                    rdma = pltpu.make_async_remote_copy(
                        src_ref=hmine,
                        dst_ref=agbuf.at[l, my_i],
                        send_sem=ag_send_sems.at[l, j],
                        recv_sem=ag_recv_sems.at[l, my_i],
                        device_id=j,
                        device_id_type=pl.DeviceIdType.LOGICAL,
                    )
                    rdma.start()

            acc2[...] = jnp.zeros((B, D), f32)
            for j in range(N_DEV):
                cp = pltpu.make_async_copy(
                    wout.at[pl.ds(j * C, C), :], wout_tile, copy_sem
                )
                cp.start()
                cp.wait()

                @pl.when(j != my_i)
                def _():
                    recv = pltpu.make_async_remote_copy(
                        src_ref=hmine,
                        dst_ref=agbuf.at[l, j],
                        send_sem=ag_send_sems.at[l, j],
                        recv_sem=ag_recv_sems.at[l, j],
                        device_id=j,
                        device_id_type=pl.DeviceIdType.LOGICAL,
                    )
                    recv.wait_recv()

                acc2[...] += jnp.dot(
                    agbuf[l, j], wout_tile[...].astype(bf16),
                    preferred_element_type=f32,
                )

            if l < 2:
                xbuf[...] = acc2[...].astype(bf16)
            else:
                out_ref[...] = acc2[...]

            for j in range(N_DEV):
                @pl.when(j != my_i)
                def _():
                    s = pltpu.make_async_remote_copy(
                        src_ref=sendbuf.at[l, j],
                        dst_ref=rsbuf.at[l, my_i],
                        send_sem=rs_send_sems.at[l, j],
                        recv_sem=rs_recv_sems.at[l, my_i],
                        device_id=j,
                        device_id_type=pl.DeviceIdType.LOGICAL,
                    )
                    s.wait_send()
                    s2 = pltpu.make_async_remote_copy(
                        src_ref=hmine,
                        dst_ref=agbuf.at[l, my_i],
                        send_sem=ag_send_sems.at[l, j],
                        recv_sem=ag_recv_sems.at[l, my_i],
                        device_id=j,
                        device_id_type=pl.DeviceIdType.LOGICAL,
                    )
                    s2.wait_send()

    return pl.pallas_call(
        body,
        out_shape=jax.ShapeDtypeStruct((B, D), f32),
        in_specs=[
            pl.BlockSpec(memory_space=pltpu.VMEM),
            pl.BlockSpec(memory_space=pltpu.HBM),
            pl.BlockSpec(memory_space=pltpu.HBM),
            pl.BlockSpec(memory_space=pltpu.HBM),
            pl.BlockSpec(memory_space=pltpu.HBM),
            pl.BlockSpec(memory_space=pltpu.HBM),
            pl.BlockSpec(memory_space=pltpu.HBM),
        ],
        out_specs=pl.BlockSpec(memory_space=pltpu.VMEM),
        scratch_shapes=[
            pltpu.VMEM((B, D), bf16),
            pltpu.VMEM((D, C), f32),
            pltpu.VMEM((C, D), f32),
            pltpu.VMEM((B, C), f32),
            pltpu.VMEM((B, D), f32),
            pltpu.VMEM((B, C), bf16),
            pltpu.VMEM((3, N_DEV, B, C), bf16),
            pltpu.VMEM((3, N_DEV, B, C), bf16),
            pltpu.VMEM((3, N_DEV, B, C), bf16),
            pltpu.SemaphoreType.DMA,
            pltpu.SemaphoreType.DMA((3, N_DEV)),
            pltpu.SemaphoreType.DMA((3, N_DEV)),
            pltpu.SemaphoreType.DMA((3, N_DEV)),
            pltpu.SemaphoreType.DMA((3, N_DEV)),
        ],
    )(x, Win0, Wout0, Win1, Wout1, Win2, Wout2)


# device time: 81117 ns/iter; 1.7008x vs baseline; 1.7008x over previous
import jax
import jax.numpy as jnp
from jax import lax
from jax.experimental import pallas as pl
from jax.experimental.pallas import tpu as pltpu

N_DEV = 8


def kernel(x, Win0, Wout0, Win1, Wout1, Win2, Wout2):
    B, D = x.shape
    H = Win0.shape[1]
    C = H // N_DEV
    f32 = jnp.float32
    bf16 = jnp.bfloat16

    def body(x_ref, win0, wout0, win1, wout1, win2, wout2, out_ref,
             xbuf, win_tiles, wout_tiles, acc, acc2, hmine,
             rsbuf, agbuf, sendbuf,
             win_sems, wout_sems,
             rs_send_sems, rs_recv_sems, ag_send_sems, ag_recv_sems):
        my_i = lax.axis_index("i")
        wins = [win0, win1, win2]
        wouts = [wout0, wout1, wout2]

        def start_win(l, j):
            pltpu.make_async_copy(
                wins[l].at[:, pl.ds(j * C, C)],
                win_tiles.at[j % 2],
                win_sems.at[j % 2],
            ).start()

        def start_wout(l, j):
            pltpu.make_async_copy(
                wouts[l].at[pl.ds(j * C, C), :],
                wout_tiles.at[j % 2],
                wout_sems.at[j % 2],
            ).start()

        xbuf[...] = x_ref[...].astype(bf16)
        start_win(0, 0)

        for l in range(3):
            for j in range(N_DEV):
                if j < N_DEV - 1:
                    start_win(l, j + 1)
                elif j == N_DEV - 1:
                    start_wout(l, 0)
                pltpu.make_async_copy(
                    wins[l].at[:, pl.ds(0, C)],
                    win_tiles.at[j % 2],
                    win_sems.at[j % 2],
                ).wait()
                chunk = jnp.dot(
                    xbuf[...], win_tiles[j % 2].astype(bf16),
                    preferred_element_type=f32,
                )

                @pl.when(j == my_i)
                def _():
                    acc[...] = chunk

                @pl.when(j != my_i)
                def _():
                    sendbuf[l, j] = chunk.astype(bf16)
                    pltpu.make_async_remote_copy(
                        src_ref=sendbuf.at[l, j],
                        dst_ref=rsbuf.at[l, my_i],
                        send_sem=rs_send_sems.at[l, j],
                        recv_sem=rs_recv_sems.at[l, my_i],
                        device_id=j,
                        device_id_type=pl.DeviceIdType.LOGICAL,
                    ).start()

            for j in range(N_DEV):
                @pl.when(j != my_i)
                def _():
                    pltpu.make_async_remote_copy(
                        src_ref=sendbuf.at[l, 0],
                        dst_ref=rsbuf.at[l, j],
                        send_sem=rs_send_sems.at[l, 0],
                        recv_sem=rs_recv_sems.at[l, j],
                        device_id=j,
                        device_id_type=pl.DeviceIdType.LOGICAL,
                    ).wait_recv()
                    acc[...] += rsbuf[l, j].astype(f32)

            hmine[...] = jnp.maximum(acc[...], 0.0).astype(bf16)

            for j in range(N_DEV):
                @pl.when(j != my_i)
                def _():
                    pltpu.make_async_remote_copy(
                        src_ref=hmine,
                        dst_ref=agbuf.at[l, my_i],
                        send_sem=ag_send_sems.at[l, j],
                        recv_sem=ag_recv_sems.at[l, my_i],
                        device_id=j,
                        device_id_type=pl.DeviceIdType.LOGICAL,
                    ).start()

            for j in range(N_DEV):
                if j < N_DEV - 1:
                    start_wout(l, j + 1)
                elif l < 2:
                    start_win(l + 1, 0)
                pltpu.make_async_copy(
                    wouts[l].at[pl.ds(0, C), :],
                    wout_tiles.at[j % 2],
                    wout_sems.at[j % 2],
                ).wait()

                @pl.when(j != my_i)
                def _():
                    pltpu.make_async_remote_copy(
                        src_ref=hmine,
                        dst_ref=agbuf.at[l, j],
                        send_sem=ag_send_sems.at[l, 0],
                        recv_sem=ag_recv_sems.at[l, j],
                        device_id=0,
                        device_id_type=pl.DeviceIdType.LOGICAL,
                    ).wait_recv()

                hj = jnp.where(j == my_i, hmine[...], agbuf[l, j])
                contrib = jnp.dot(
                    hj, wout_tiles[j % 2].astype(bf16),
                    preferred_element_type=f32,
                )
                if j == 0:
                    acc2[...] = contrib
                else:
                    acc2[...] += contrib

            if l < 2:
                xbuf[...] = acc2[...].astype(bf16)
            else:
                out_ref[...] = acc2[...]

            for j in range(N_DEV):
                @pl.when(j != my_i)
                def _():
                    pltpu.make_async_remote_copy(
                        src_ref=sendbuf.at[l, j],
                        dst_ref=rsbuf.at[l, my_i],
                        send_sem=rs_send_sems.at[l, j],
                        recv_sem=rs_recv_sems.at[l, my_i],
                        device_id=0,
                        device_id_type=pl.DeviceIdType.LOGICAL,
                    ).wait_send()
                    pltpu.make_async_remote_copy(
                        src_ref=hmine,
                        dst_ref=agbuf.at[l, my_i],
                        send_sem=ag_send_sems.at[l, j],
                        recv_sem=ag_recv_sems.at[l, my_i],
                        device_id=j,
                        device_id_type=pl.DeviceIdType.LOGICAL,
                    ).wait_send()

    return pl.pallas_call(
        body,
        out_shape=jax.ShapeDtypeStruct((B, D), f32),
        in_specs=[
            pl.BlockSpec(memory_space=pltpu.VMEM),
            pl.BlockSpec(memory_space=pltpu.HBM),
            pl.BlockSpec(memory_space=pltpu.HBM),
            pl.BlockSpec(memory_space=pltpu.HBM),
            pl.BlockSpec(memory_space=pltpu.HBM),
            pl.BlockSpec(memory_space=pltpu.HBM),
            pl.BlockSpec(memory_space=pltpu.HBM),
        ],
        out_specs=pl.BlockSpec(memory_space=pltpu.VMEM),
        scratch_shapes=[
            pltpu.VMEM((B, D), bf16),
            pltpu.VMEM((2, D, C), f32),
            pltpu.VMEM((2, C, D), f32),
            pltpu.VMEM((B, C), f32),
            pltpu.VMEM((B, D), f32),
            pltpu.VMEM((B, C), bf16),
            pltpu.VMEM((3, N_DEV, B, C), bf16),
            pltpu.VMEM((3, N_DEV, B, C), bf16),
            pltpu.VMEM((3, N_DEV, B, C), bf16),
            pltpu.SemaphoreType.DMA((2,)),
            pltpu.SemaphoreType.DMA((2,)),
            pltpu.SemaphoreType.DMA((3, N_DEV)),
            pltpu.SemaphoreType.DMA((3, N_DEV)),
            pltpu.SemaphoreType.DMA((3, N_DEV)),
            pltpu.SemaphoreType.DMA((3, N_DEV)),
        ],
    )(x, Win0, Wout0, Win1, Wout1, Win2, Wout2)
